# Initial kernel scaffold; baseline (speedup 1.0000x reference)
import jax
import jax.numpy as jnp
from jax.experimental import pallas as pl


def kernel(x, edge_index, W1, b1, W2, b2):
    raise NotImplementedError("write your pallas kernel here")


# speedup vs baseline: 10.4459x; 10.4459x over previous
import functools

import jax
import jax.numpy as jnp
from jax import lax
from jax.experimental import pallas as pl
from jax.experimental.pallas import tpu as pltpu
from jax.experimental.pallas import tpu_sc as plsc

_N = 10000
_E = 320000
_DIN = 128
_DH = 16
_DOUT = 128

_NC = 2
_NS = 16
_NW = _NC * _NS
_EPW = 10240
_EP = _NW * _EPW
_IDX_W = 128
_ROWS_PW = _EPW // _IDX_W
_CHUNK_ROWS = 8
_NCHUNK = _ROWS_PW // _CHUNK_ROWS
_DUMP = _N
_ACC_ROWS = 10240
_TBL_ROWS = _N + 16
_ZSL = _ACC_ROWS // _NS

_f32 = jnp.float32


def _deg_body(src_hbm, dst_hbm, out_hbm, idxs_v, idxd_v, ones_v, zero_v,
              acc_o, acc_i):
  cid = lax.axis_index("c")
  sid = lax.axis_index("s")
  wid = cid * _NS + sid

  def _fill_z(i, c):
    zero_v[i] = jnp.zeros((16,), _f32)
    return c

  lax.fori_loop(0, _ZSL, _fill_z, 0)

  def _fill_o(i, c):
    ones_v[i] = jnp.ones((16,), _f32)
    return c

  lax.fori_loop(0, _IDX_W, _fill_o, 0)

  pltpu.sync_copy(zero_v, acc_o.at[pl.ds(sid * _ZSL, _ZSL)])
  pltpu.sync_copy(zero_v, acc_i.at[pl.ds(sid * _ZSL, _ZSL)])
  plsc.subcore_barrier()

  base = wid * _ROWS_PW

  def _chunk(c, carry):
    r0 = base + c * _CHUNK_ROWS
    pltpu.sync_copy(src_hbm.at[pl.ds(r0, _CHUNK_ROWS)], idxs_v)
    pltpu.sync_copy(dst_hbm.at[pl.ds(r0, _CHUNK_ROWS)], idxd_v)
    for j in range(_CHUNK_ROWS):
      pltpu.sync_copy(ones_v, acc_o.at[idxs_v.at[j]], add=True)
      pltpu.sync_copy(ones_v, acc_i.at[idxd_v.at[j]], add=True)
    return carry

  lax.fori_loop(0, _NCHUNK, _chunk, 0)
  plsc.subcore_barrier()

  pltpu.sync_copy(acc_o.at[pl.ds(sid * _ZSL, _ZSL)],
                  out_hbm.at[cid, 0, pl.ds(sid * _ZSL, _ZSL)])
  pltpu.sync_copy(acc_i.at[pl.ds(sid * _ZSL, _ZSL)],
                  out_hbm.at[cid, 1, pl.ds(sid * _ZSL, _ZSL)])


_deg_call = pl.kernel(
    _deg_body,
    out_type=jax.ShapeDtypeStruct((_NC, 2, _ACC_ROWS, _DH), _f32),
    mesh=plsc.VectorSubcoreMesh(core_axis_name="c", subcore_axis_name="s"),
    scratch_types=[
        pltpu.VMEM((_CHUNK_ROWS, _IDX_W), jnp.int32),
        pltpu.VMEM((_CHUNK_ROWS, _IDX_W), jnp.int32),
        pltpu.VMEM((_IDX_W, _DH), _f32),
        pltpu.VMEM((_ZSL, _DH), _f32),
        pltpu.VMEM_SHARED((_ACC_ROWS, _DH), _f32),
        pltpu.VMEM_SHARED((_ACC_ROWS, _DH), _f32),
    ],
    compiler_params=pltpu.CompilerParams(use_tc_tiling_on_sc=False),
)


def _agg_body(h_hbm, src_hbm, dst_hbm, out_hbm, idxs_v, idxd_v, rows_v,
              zero_v, acc, sem):
  cid = lax.axis_index("c")
  sid = lax.axis_index("s")
  wid = cid * _NS + sid

  def _fill_z(i, c):
    zero_v[i] = jnp.zeros((16,), _f32)
    return c

  lax.fori_loop(0, _ZSL, _fill_z, 0)
  pltpu.sync_copy(zero_v, acc.at[pl.ds(sid * _ZSL, _ZSL)])
  plsc.subcore_barrier()

  base = wid * _ROWS_PW

  def _chunk(c, carry):
    r0 = base + c * _CHUNK_ROWS
    pltpu.sync_copy(src_hbm.at[pl.ds(r0, _CHUNK_ROWS)], idxs_v)
    pltpu.sync_copy(dst_hbm.at[pl.ds(r0, _CHUNK_ROWS)], idxd_v)
    cps = [
        pltpu.async_copy(h_hbm.at[idxs_v.at[j]],
                         rows_v.at[pl.ds(j * _IDX_W, _IDX_W)], sem)
        for j in range(_CHUNK_ROWS)
    ]
    for cp in cps:
      cp.wait()
    for j in range(_CHUNK_ROWS):
      pltpu.sync_copy(rows_v.at[pl.ds(j * _IDX_W, _IDX_W)],
                      acc.at[idxd_v.at[j]], add=True)
    return carry

  lax.fori_loop(0, _NCHUNK, _chunk, 0)
  plsc.subcore_barrier()

  pltpu.sync_copy(acc.at[pl.ds(sid * _ZSL, _ZSL)],
                  out_hbm.at[cid, pl.ds(sid * _ZSL, _ZSL)])


_agg_call = pl.kernel(
    _agg_body,
    out_type=jax.ShapeDtypeStruct((_NC, _ACC_ROWS, _DH), _f32),
    mesh=plsc.VectorSubcoreMesh(core_axis_name="c", subcore_axis_name="s"),
    scratch_types=[
        pltpu.VMEM((_CHUNK_ROWS, _IDX_W), jnp.int32),
        pltpu.VMEM((_CHUNK_ROWS, _IDX_W), jnp.int32),
        pltpu.VMEM((_CHUNK_ROWS * _IDX_W, _DH), _f32),
        pltpu.VMEM((_ZSL, _DH), _f32),
        pltpu.VMEM_SHARED((_ACC_ROWS, _DH), _f32),
        pltpu.SemaphoreType.DMA,
    ],
    compiler_params=pltpu.CompilerParams(use_tc_tiling_on_sc=False),
)


_GRID = 10
_BR = _N // _GRID


def _norms(deg_ref):
  deg_o = deg_ref[0, 0] + deg_ref[1, 0]
  deg_i = deg_ref[0, 1] + deg_ref[1, 1]
  ns = jnp.where(deg_o > 0, lax.rsqrt(jnp.maximum(deg_o, 1.0)), 1.0)
  nd = jnp.where(deg_i > 0, lax.rsqrt(jnp.maximum(deg_i, 1.0)), 1.0)
  return ns, nd


def _mm1_body(x_ref, w_ref, o_ref):
  o_ref[...] = jnp.dot(x_ref[...], w_ref[...], preferred_element_type=_f32)


_mm1_call = pl.pallas_call(
    _mm1_body,
    grid=(_GRID,),
    in_specs=[
        pl.BlockSpec((_BR, _DIN), lambda i: (i, 0)),
        pl.BlockSpec((_DIN, _DH), lambda i: (0, 0)),
    ],
    out_specs=pl.BlockSpec((_BR, _DH), lambda i: (i, 0)),
    out_shape=jax.ShapeDtypeStruct((_N, _DH), _f32),
)


def _scale_body(xw_ref, deg_ref, o_ref):
  ns, _ = _norms(deg_ref)
  o_ref[...] = xw_ref[...] * ns


_scale_call = pl.pallas_call(
    _scale_body,
    grid=(_GRID,),
    in_specs=[
        pl.BlockSpec((_BR, _DH), lambda i: (i, 0)),
        pl.BlockSpec((_NC, 2, _BR, _DH), lambda i: (0, 0, i, 0)),
    ],
    out_specs=pl.BlockSpec((_BR, _DH), lambda i: (i, 0)),
    out_shape=jax.ShapeDtypeStruct((_N, _DH), _f32),
)


def _mid_body(a_ref, deg_ref, b_ref, o_ref):
  a = a_ref[0] + a_ref[1]
  ns, nd = _norms(deg_ref)
  h = jnp.maximum(a * nd + b_ref[...], 0.0)
  o_ref[...] = h * ns


_mid_call = pl.pallas_call(
    _mid_body,
    grid=(_GRID,),
    in_specs=[
        pl.BlockSpec((_NC, _BR, _DH), lambda i: (0, i, 0)),
        pl.BlockSpec((_NC, 2, _BR, _DH), lambda i: (0, 0, i, 0)),
        pl.BlockSpec((1, _DH), lambda i: (0, 0)),
    ],
    out_specs=pl.BlockSpec((_BR, _DH), lambda i: (i, 0)),
    out_shape=jax.ShapeDtypeStruct((_N, _DH), _f32),
)


def _fin_body(a_ref, deg_ref, w_ref, b_ref, o_ref):
  a = a_ref[0] + a_ref[1]
  _, nd = _norms(deg_ref)
  o_ref[...] = (
      jnp.dot(a * nd, w_ref[...], preferred_element_type=_f32) + b_ref[...])


_fin_call = pl.pallas_call(
    _fin_body,
    grid=(_GRID,),
    in_specs=[
        pl.BlockSpec((_NC, _BR, _DH), lambda i: (0, i, 0)),
        pl.BlockSpec((_NC, 2, _BR, _DH), lambda i: (0, 0, i, 0)),
        pl.BlockSpec((_DH, _DOUT), lambda i: (0, 0)),
        pl.BlockSpec((1, _DOUT), lambda i: (0, 0)),
    ],
    out_specs=pl.BlockSpec((_BR, _DOUT), lambda i: (i, 0)),
    out_shape=jax.ShapeDtypeStruct((_N, _DOUT), _f32),
)


@jax.jit
def kernel(x, edge_index, W1, b1, W2, b2):
  src = edge_index[0].astype(jnp.int32)
  dst = edge_index[1].astype(jnp.int32)
  pad = _EP - _E
  padv = jnp.full((pad,), _DUMP, jnp.int32)
  srcp = jnp.concatenate([src, padv]).reshape(_EP // _IDX_W, _IDX_W)
  dstp = jnp.concatenate([dst, padv]).reshape(_EP // _IDX_W, _IDX_W)

  degp = _deg_call(srcp, dstp)
  xw1 = _mm1_call(x, W1)
  h1s = _scale_call(xw1, degp)
  h1t = jnp.pad(h1s, ((0, _TBL_ROWS - _N), (0, 0)))
  a1p = _agg_call(h1t, srcp, dstp)
  h2s = _mid_call(a1p, degp, b1.reshape(1, _DH))
  h2t = jnp.pad(h2s, ((0, _TBL_ROWS - _N), (0, 0)))
  a2p = _agg_call(h2t, srcp, dstp)
  return _fin_call(a2p, degp, W2, b2.reshape(1, _DOUT))


# speedup vs baseline: 13.1215x; 1.2561x over previous
import jax
import jax.numpy as jnp
from jax import lax
from jax.experimental import pallas as pl
from jax.experimental.pallas import tpu as pltpu
from jax.experimental.pallas import tpu_sc as plsc

_N = 10000
_E = 320000
_DIN = 128
_DH = 16
_DOUT = 128

_NC = 2
_NS = 16
_NW = _NC * _NS
_EPW = 10240
_EP = _NW * _EPW
_IDX_W = 128
_ROWS_PW = _EPW // _IDX_W
_CHUNK_ROWS = 8
_NCHUNK = _ROWS_PW // _CHUNK_ROWS
_CR = _CHUNK_ROWS * _IDX_W
_DUMP = _N
_ACC_ROWS = 10240
_TBL_ROWS = _N + 16
_ZSL = _ACC_ROWS // _NS

_f32 = jnp.float32


def _fill(ref, n, vec):
  def body(i, c):
    ref[i] = vec
    return c

  lax.fori_loop(0, n, body, 0)


def _deg_body(src_hbm, dst_hbm, out_hbm, idxs_all, idxd_all, ones_v, zero_v,
              acc_o, acc_i, sem_o, sem_i):
  cid = lax.axis_index("c")
  sid = lax.axis_index("s")
  wid = cid * _NS + sid

  _fill(zero_v, _ZSL, jnp.zeros((16,), _f32))
  _fill(ones_v, _IDX_W, jnp.ones((16,), _f32))
  pltpu.sync_copy(zero_v, acc_o.at[pl.ds(sid * _ZSL, _ZSL)])
  pltpu.sync_copy(zero_v, acc_i.at[pl.ds(sid * _ZSL, _ZSL)])
  base = wid * _ROWS_PW
  pltpu.sync_copy(src_hbm.at[pl.ds(base, _ROWS_PW)], idxs_all)
  pltpu.sync_copy(dst_hbm.at[pl.ds(base, _ROWS_PW)], idxd_all)
  plsc.subcore_barrier()

  def _chunk(c, carry):
    r0 = c * _CHUNK_ROWS
    for j in range(_CHUNK_ROWS):
      pltpu.async_copy(ones_v, acc_o.at[idxs_all.at[r0 + j]], sem_o, add=True)
      pltpu.async_copy(ones_v, acc_i.at[idxd_all.at[r0 + j]], sem_i, add=True)
    for j in range(_CHUNK_ROWS):
      pltpu.make_async_copy(ones_v, acc_o.at[idxs_all.at[r0 + j]], sem_o).wait()
      pltpu.make_async_copy(ones_v, acc_i.at[idxd_all.at[r0 + j]], sem_i).wait()
    return carry

  lax.fori_loop(0, _NCHUNK, _chunk, 0)
  plsc.subcore_barrier()

  pltpu.sync_copy(acc_o.at[pl.ds(sid * _ZSL, _ZSL)],
                  out_hbm.at[cid, 0, pl.ds(sid * _ZSL, _ZSL)])
  pltpu.sync_copy(acc_i.at[pl.ds(sid * _ZSL, _ZSL)],
                  out_hbm.at[cid, 1, pl.ds(sid * _ZSL, _ZSL)])


_deg_call = pl.kernel(
    _deg_body,
    out_type=jax.ShapeDtypeStruct((_NC, 2, _ACC_ROWS, _DH), _f32),
    mesh=plsc.VectorSubcoreMesh(core_axis_name="c", subcore_axis_name="s"),
    scratch_types=[
        pltpu.VMEM((_ROWS_PW, _IDX_W), jnp.int32),
        pltpu.VMEM((_ROWS_PW, _IDX_W), jnp.int32),
        pltpu.VMEM((_IDX_W, _DH), _f32),
        pltpu.VMEM((_ZSL, _DH), _f32),
        pltpu.VMEM_SHARED((_ACC_ROWS, _DH), _f32),
        pltpu.VMEM_SHARED((_ACC_ROWS, _DH), _f32),
        pltpu.SemaphoreType.DMA,
        pltpu.SemaphoreType.DMA,
    ],
    compiler_params=pltpu.CompilerParams(use_tc_tiling_on_sc=False),
)


def _agg_body(h_hbm, src_hbm, dst_hbm, out_hbm, idxs_all, idxd_all, r_a, r_b,
              zero_v, acc, sem_a, sem_b):
  cid = lax.axis_index("c")
  sid = lax.axis_index("s")
  wid = cid * _NS + sid

  _fill(zero_v, _ZSL, jnp.zeros((16,), _f32))
  pltpu.sync_copy(zero_v, acc.at[pl.ds(sid * _ZSL, _ZSL)])
  base = wid * _ROWS_PW
  pltpu.sync_copy(src_hbm.at[pl.ds(base, _ROWS_PW)], idxs_all)
  pltpu.sync_copy(dst_hbm.at[pl.ds(base, _ROWS_PW)], idxd_all)
  plsc.subcore_barrier()

  def _fire(chunk_row0, rows, sem):
    for j in range(_CHUNK_ROWS):
      row = jnp.minimum(chunk_row0 + j, _ROWS_PW - 1)
      pltpu.async_copy(h_hbm.at[idxs_all.at[row]],
                       rows.at[pl.ds(j * _IDX_W, _IDX_W)], sem)

  def _drain(rows, sem):
    pltpu.make_async_copy(h_hbm.at[pl.ds(0, _CR)], rows, sem).wait()

  def _scatter(chunk_row0, rows):
    for j in range(_CHUNK_ROWS):
      pltpu.sync_copy(rows.at[pl.ds(j * _IDX_W, _IDX_W)],
                      acc.at[idxd_all.at[chunk_row0 + j]], add=True)

  _fire(0, r_a, sem_a)

  def _pair(p, carry):
    r0a = 2 * p * _CHUNK_ROWS
    r0b = r0a + _CHUNK_ROWS
    _fire(r0b, r_b, sem_b)
    _drain(r_a, sem_a)
    _scatter(r0a, r_a)
    _fire(r0b + _CHUNK_ROWS, r_a, sem_a)
    _drain(r_b, sem_b)
    _scatter(r0b, r_b)
    return carry

  lax.fori_loop(0, _NCHUNK // 2, _pair, 0)
  _drain(r_a, sem_a)
  plsc.subcore_barrier()

  pltpu.sync_copy(acc.at[pl.ds(sid * _ZSL, _ZSL)],
                  out_hbm.at[cid, pl.ds(sid * _ZSL, _ZSL)])


_agg_call = pl.kernel(
    _agg_body,
    out_type=jax.ShapeDtypeStruct((_NC, _ACC_ROWS, _DH), _f32),
    mesh=plsc.VectorSubcoreMesh(core_axis_name="c", subcore_axis_name="s"),
    scratch_types=[
        pltpu.VMEM((_ROWS_PW, _IDX_W), jnp.int32),
        pltpu.VMEM((_ROWS_PW, _IDX_W), jnp.int32),
        pltpu.VMEM((_CR, _DH), _f32),
        pltpu.VMEM((_CR, _DH), _f32),
        pltpu.VMEM((_ZSL, _DH), _f32),
        pltpu.VMEM_SHARED((_ACC_ROWS, _DH), _f32),
        pltpu.SemaphoreType.DMA,
        pltpu.SemaphoreType.DMA,
    ],
    compiler_params=pltpu.CompilerParams(use_tc_tiling_on_sc=False),
)


_GRID = 10
_BR = _N // _GRID


def _norms(deg_ref):
  deg_o = deg_ref[0, 0] + deg_ref[1, 0]
  deg_i = deg_ref[0, 1] + deg_ref[1, 1]
  ns = jnp.where(deg_o > 0, lax.rsqrt(jnp.maximum(deg_o, 1.0)), 1.0)
  nd = jnp.where(deg_i > 0, lax.rsqrt(jnp.maximum(deg_i, 1.0)), 1.0)
  return ns, nd


def _l1_body(x_ref, w_ref, deg_ref, o_ref):
  ns, _ = _norms(deg_ref)
  o_ref[...] = jnp.dot(x_ref[...], w_ref[...], preferred_element_type=_f32) * ns


_l1_call = pl.pallas_call(
    _l1_body,
    grid=(_GRID,),
    in_specs=[
        pl.BlockSpec((_BR, _DIN), lambda i: (i, 0)),
        pl.BlockSpec((_DIN, _DH), lambda i: (0, 0)),
        pl.BlockSpec((_NC, 2, _BR, _DH), lambda i: (0, 0, i, 0)),
    ],
    out_specs=pl.BlockSpec((_BR, _DH), lambda i: (i, 0)),
    out_shape=jax.ShapeDtypeStruct((_TBL_ROWS, _DH), _f32),
)


def _mid_body(a_ref, deg_ref, b_ref, o_ref):
  a = a_ref[0] + a_ref[1]
  ns, nd = _norms(deg_ref)
  h = jnp.maximum(a * nd + b_ref[...], 0.0)
  o_ref[...] = h * ns


_mid_call = pl.pallas_call(
    _mid_body,
    grid=(_GRID,),
    in_specs=[
        pl.BlockSpec((_NC, _BR, _DH), lambda i: (0, i, 0)),
        pl.BlockSpec((_NC, 2, _BR, _DH), lambda i: (0, 0, i, 0)),
        pl.BlockSpec((1, _DH), lambda i: (0, 0)),
    ],
    out_specs=pl.BlockSpec((_BR, _DH), lambda i: (i, 0)),
    out_shape=jax.ShapeDtypeStruct((_TBL_ROWS, _DH), _f32),
)


def _fin_body(a_ref, deg_ref, w_ref, b_ref, o_ref):
  a = a_ref[0] + a_ref[1]
  _, nd = _norms(deg_ref)
  o_ref[...] = (
      jnp.dot(a * nd, w_ref[...], preferred_element_type=_f32) + b_ref[...])


_fin_call = pl.pallas_call(
    _fin_body,
    grid=(_GRID,),
    in_specs=[
        pl.BlockSpec((_NC, _BR, _DH), lambda i: (0, i, 0)),
        pl.BlockSpec((_NC, 2, _BR, _DH), lambda i: (0, 0, i, 0)),
        pl.BlockSpec((_DH, _DOUT), lambda i: (0, 0)),
        pl.BlockSpec((1, _DOUT), lambda i: (0, 0)),
    ],
    out_specs=pl.BlockSpec((_BR, _DOUT), lambda i: (i, 0)),
    out_shape=jax.ShapeDtypeStruct((_N, _DOUT), _f32),
)


@jax.jit
def kernel(x, edge_index, W1, b1, W2, b2):
  src = edge_index[0].astype(jnp.int32)
  dst = edge_index[1].astype(jnp.int32)
  pad = _EP - _E
  padv = jnp.full((pad,), _DUMP, jnp.int32)
  srcp = jnp.concatenate([src, padv]).reshape(_EP // _IDX_W, _IDX_W)
  dstp = jnp.concatenate([dst, padv]).reshape(_EP // _IDX_W, _IDX_W)

  degp = _deg_call(srcp, dstp)
  h1t = _l1_call(x, W1, degp)
  a1p = _agg_call(h1t, srcp, dstp)
  h2t = _mid_call(a1p, degp, b1.reshape(1, _DH))
  a2p = _agg_call(h2t, srcp, dstp)
  return _fin_call(a2p, degp, W2, b2.reshape(1, _DOUT))
